# baseline (speedup 1.0000x reference)
import functools

import jax
import jax.numpy as jnp
from jax import lax
from jax.experimental import pallas as pl
from jax.experimental.pallas import tpu as pltpu
from jax.experimental.pallas import tpu_sc as plsc

BATCH = 8192
D = 2048
L = 16
NC = 2
NS = 16
NW = NC * NS
ROWS_PER_W = BATCH // NW
CHUNK = 16
N_CHUNKS = ROWS_PER_W // CHUNK
JBLKS = D // L


def _body(tgt_hbm, perm_hbm, inv_hbm, flag_hbm, out_hbm,
          perm_v, inv_v, flag_v, sel_v, in_v, out_v):
  wid = lax.axis_index("s") * NC + lax.axis_index("c")

  pltpu.sync_copy(perm_hbm, perm_v)
  pltpu.sync_copy(inv_hbm, inv_v)
  pltpu.sync_copy(flag_hbm, flag_v)
  use_inv = flag_v[...] != 0

  def sel_body(j, carry):
    p = perm_v[pl.ds(j * L, L)]
    q = inv_v[pl.ds(j * L, L)]
    sel_v[pl.ds(j * L, L)] = lax.select(use_inv, q, p)
    return carry

  lax.fori_loop(0, JBLKS, sel_body, 0)

  base = wid * ROWS_PER_W
  for c in range(N_CHUNKS):
    elem0 = (base + c * CHUNK) * D
    pltpu.sync_copy(tgt_hbm.at[pl.ds(elem0, CHUNK * D)], in_v)

    def j_body(j, carry):
      col = sel_v[pl.ds(j * L, L)]
      off = j * L
      for r in range(CHUNK):
        vals = plsc.load_gather(in_v, [col + (r * D)])
        out_v[pl.ds(r * D + off, L)] = vals
      return carry

    lax.fori_loop(0, JBLKS, j_body, 0)
    pltpu.sync_copy(out_v, out_hbm.at[pl.ds(elem0, CHUNK * D)])


@functools.partial(
    pl.kernel,
    mesh=plsc.VectorSubcoreMesh(core_axis_name="c", subcore_axis_name="s"),
    out_type=jax.ShapeDtypeStruct((BATCH * D,), jnp.float32),
    compiler_params=pltpu.CompilerParams(
        needs_layout_passes=False, use_tc_tiling_on_sc=False),
    scratch_types=[
        pltpu.VMEM((D,), jnp.int32),
        pltpu.VMEM((D,), jnp.int32),
        pltpu.VMEM((L,), jnp.int32),
        pltpu.VMEM((D,), jnp.int32),
        pltpu.VMEM((CHUNK * D,), jnp.float32),
        pltpu.VMEM((CHUNK * D,), jnp.float32),
    ],
)
def _permute_sc(tgt_hbm, perm_hbm, inv_hbm, flag_hbm, out_hbm,
                perm_v, inv_v, flag_v, sel_v, in_v, out_v):
  _body(tgt_hbm, perm_hbm, inv_hbm, flag_hbm, out_hbm,
        perm_v, inv_v, flag_v, sel_v, in_v, out_v)


@jax.jit
def kernel(target, permutation, inv_permutation, inverse):
  flag = jnp.broadcast_to(jnp.asarray(inverse, jnp.int32), (L,))
  flat = _permute_sc(target.reshape(BATCH * D), permutation,
                     inv_permutation, flag)
  return flat.reshape(BATCH, D)


# speedup vs baseline: 1.7173x; 1.7173x over previous
import functools

import jax
import jax.numpy as jnp
from jax import lax
from jax.experimental import pallas as pl
from jax.experimental.pallas import tpu as pltpu
from jax.experimental.pallas import tpu_sc as plsc

BATCH = 8192
D = 2048
L = 16
NC = 2
NS = 16
NW = NC * NS
ROWS_PER_W = BATCH // NW
CHUNK = 8
N_CHUNKS = ROWS_PER_W // CHUNK
JBLKS = D // L
NBUF = 2


def _gather_chunk(in_ref, out_ref, sel_v):
  @plsc.parallel_loop(0, JBLKS, unroll=2)
  def _(j):
    col = sel_v[pl.ds(j * L, L)]
    off = j * L
    for r in range(CHUNK):
      vals = plsc.load_gather(in_ref.at[pl.ds(r * D, D)], [col])
      out_ref[pl.ds(r * D + off, L)] = vals


def _body(tgt_hbm, perm_hbm, inv_hbm, flag_hbm, out_hbm,
          perm_v, inv_v, flag_v, sel_v, in_v, out_v, sems):
  wid = lax.axis_index("s") * NC + lax.axis_index("c")

  pltpu.sync_copy(perm_hbm, perm_v)
  pltpu.sync_copy(inv_hbm, inv_v)
  pltpu.sync_copy(flag_hbm, flag_v)
  use_inv = flag_v[...] != 0

  def sel_body(j, carry):
    p = perm_v[pl.ds(j * L, L)]
    q = inv_v[pl.ds(j * L, L)]
    sel_v[pl.ds(j * L, L)] = lax.select(use_inv, q, p)
    return carry

  lax.fori_loop(0, JBLKS, sel_body, 0)

  base = wid * ROWS_PER_W

  def chunk_slice(c):
    return pl.ds((base + c * CHUNK) * D, CHUNK * D)

  h_in = [None] * NBUF
  h_out = [None] * NBUF
  h_in[0] = pltpu.async_copy(tgt_hbm.at[chunk_slice(0)], in_v.at[0],
                             sems.at[0])
  for c in range(N_CHUNKS):
    b = c % NBUF
    nb = (c + 1) % NBUF
    if c + 1 < N_CHUNKS:
      h_in[nb] = pltpu.async_copy(tgt_hbm.at[chunk_slice(c + 1)],
                                  in_v.at[nb], sems.at[nb])
    if c >= NBUF:
      h_out[b].wait()
    h_in[b].wait()
    _gather_chunk(in_v.at[b], out_v.at[b], sel_v)
    h_out[b] = pltpu.async_copy(out_v.at[b], out_hbm.at[chunk_slice(c)],
                                sems.at[NBUF + b])
  for b in range(NBUF):
    h_out[b].wait()


@functools.partial(
    pl.kernel,
    mesh=plsc.VectorSubcoreMesh(core_axis_name="c", subcore_axis_name="s"),
    out_type=jax.ShapeDtypeStruct((BATCH * D,), jnp.float32),
    compiler_params=pltpu.CompilerParams(
        needs_layout_passes=False, use_tc_tiling_on_sc=False),
    scratch_types=[
        pltpu.VMEM((D,), jnp.int32),
        pltpu.VMEM((D,), jnp.int32),
        pltpu.VMEM((L,), jnp.int32),
        pltpu.VMEM((D,), jnp.int32),
        pltpu.VMEM((NBUF, CHUNK * D), jnp.float32),
        pltpu.VMEM((NBUF, CHUNK * D), jnp.float32),
        pltpu.SemaphoreType.DMA((2 * NBUF,)),
    ],
)
def _permute_sc(tgt_hbm, perm_hbm, inv_hbm, flag_hbm, out_hbm,
                perm_v, inv_v, flag_v, sel_v, in_v, out_v, sems):
  _body(tgt_hbm, perm_hbm, inv_hbm, flag_hbm, out_hbm,
        perm_v, inv_v, flag_v, sel_v, in_v, out_v, sems)


@jax.jit
def kernel(target, permutation, inv_permutation, inverse):
  flag = jnp.broadcast_to(jnp.asarray(inverse, jnp.int32), (L,))
  flat = _permute_sc(target.reshape(BATCH * D), permutation,
                     inv_permutation, flag)
  return flat.reshape(BATCH, D)
